# Initial kernel scaffold; baseline (speedup 1.0000x reference)
import jax
import jax.numpy as jnp
from jax.experimental import pallas as pl


def kernel(x, edge_index, W, a):
    raise NotImplementedError("write your pallas kernel here")


# speedup vs baseline: 7.5093x; 7.5093x over previous
import functools

import jax
import jax.numpy as jnp
from jax import lax
from jax.experimental import pallas as pl
from jax.experimental.pallas import tpu as pltpu
from jax.experimental.pallas import tpu_sc as plsc

EPS = 1e-15
MAXNORM = 1.0 - 1e-5

N = 10000
E = 160000
D = 256
H = 128
NC = 2
NS = 16
EPT = E // NS
K = 80
CH = EPT // K
RPT = 624
TAIL = N - NS * RPT
BT = 400


def _artanh(z):
    z = jnp.clip(z, -1.0 + 1e-7, 1.0 - 1e-7)
    return 0.5 * jnp.log((1.0 + z) / (1.0 - z))


def _prepass_body(x_ref, w_ref, a_ref, t2_ref, s1_ref, s2_ref):
    x = x_ref[...]
    wm = w_ref[...]
    a = a_ref[...]
    mx = lax.dot_general(x, wm, (((1,), (1,)), ((), ())),
                         preferred_element_type=jnp.float32)
    xn = jnp.maximum(jnp.sqrt(jnp.sum(x * x, axis=1, keepdims=True)), EPS)
    mxn = jnp.maximum(jnp.sqrt(jnp.sum(mx * mx, axis=1, keepdims=True)), EPS)
    th = jnp.tanh(mxn / xn * _artanh(xn))
    h = mx * (th / mxn)
    h = jnp.where(th > MAXNORM, h * (MAXNORM / th), h)
    hn = jnp.maximum(jnp.minimum(th, MAXNORM), EPS)
    t = h * (_artanh(hn) / hn)
    s1 = jnp.sum(t * a[0, :D][None, :], axis=1)
    s2 = jnp.sum(t * a[0, D:][None, :], axis=1)
    t2_ref[0] = t[:, :H]
    t2_ref[1] = t[:, H:]
    s1_ref[...] = s1[:, None]
    s2_ref[...] = s2[:, None]


@jax.jit
def _prepass(x, wmat, a):
    grid = (N // BT,)
    return pl.pallas_call(
        _prepass_body,
        grid=grid,
        in_specs=[
            pl.BlockSpec((BT, D), lambda i: (i, 0)),
            pl.BlockSpec((D, D), lambda i: (0, 0)),
            pl.BlockSpec((1, 2 * D), lambda i: (0, 0)),
        ],
        out_specs=[
            pl.BlockSpec((2, BT, H), lambda i: (0, i, 0)),
            pl.BlockSpec((BT, 1), lambda i: (i, 0)),
            pl.BlockSpec((BT, 1), lambda i: (i, 0)),
        ],
        out_shape=[
            jax.ShapeDtypeStruct((2, N, H), jnp.float32),
            jax.ShapeDtypeStruct((N, 1), jnp.float32),
            jax.ShapeDtypeStruct((N, 1), jnp.float32),
        ],
    )(x, wmat, a)


def _wsum_body(s1_hbm, s2_hbm, src_hbm, dst_hbm,
               w_hbm, parts_hbm,
               s1_v, s2_v, srcs_v, dsts_v, wv_v, rsum_v):
    c = lax.axis_index("c")
    s = lax.axis_index("s")

    pltpu.sync_copy(s1_hbm, s1_v)
    pltpu.sync_copy(s2_hbm, s2_v)
    pltpu.sync_copy(src_hbm.at[s], srcs_v)
    pltpu.sync_copy(dst_hbm.at[s], dsts_v)

    z16 = jnp.zeros((16,), jnp.float32)

    def _zero(g, _):
        rsum_v[pl.ds(g * 16, 16)] = z16
        return 0
    lax.fori_loop(0, N // 16, _zero, 0)

    def _row(j, _):
        for k in range(K // 16):
            sl = pl.ds(k * 16, 16)
            src16 = srcs_v[j, sl]
            sv = (plsc.load_gather(s1_v, [src16])
                  + plsc.load_gather(s2_v, [dsts_v[j, sl]]))
            lr = jnp.maximum(sv, 0.0) + 0.2 * jnp.minimum(sv, 0.0)
            w16 = jnp.exp(-lr)
            wv_v[j, sl] = w16
            plsc.addupdate_scatter(rsum_v, [src16], w16)
        return 0
    lax.fori_loop(0, CH, _row, 0)

    @pl.when(c == 0)
    def _():
        pltpu.sync_copy(wv_v, w_hbm.at[s])
        pltpu.sync_copy(rsum_v, parts_hbm.at[s])


@jax.jit
def _wsum_pass(s1, s2, src3, dst3):
    mesh = plsc.VectorSubcoreMesh(core_axis_name="c", subcore_axis_name="s",
                                  num_cores=NC, num_subcores=NS)
    return pl.kernel(
        _wsum_body,
        out_type=(
            jax.ShapeDtypeStruct((NS, CH, K), jnp.float32),
            jax.ShapeDtypeStruct((NS, N), jnp.float32),
        ),
        mesh=mesh,
        compiler_params=pltpu.CompilerParams(needs_layout_passes=False),
        scratch_types=[
            pltpu.VMEM((N,), jnp.float32),
            pltpu.VMEM((N,), jnp.float32),
            pltpu.VMEM((CH, K), jnp.int32),
            pltpu.VMEM((CH, K), jnp.int32),
            pltpu.VMEM((CH, K), jnp.float32),
            pltpu.VMEM((N,), jnp.float32),
        ],
    )(s1, s2, src3, dst3)


def _scatter_body(t2_hbm, w_hbm, src_hbm, dst_hbm,
                  hp_hbm,
                  srcs_v, dsts_v, wch_v, rows_v,
                  hp_sh):
    c = lax.axis_index("c")
    s = lax.axis_index("s")
    coff = c * N

    pltpu.sync_copy(src_hbm.at[s], srcs_v)
    pltpu.sync_copy(dst_hbm.at[s], dsts_v)

    z16 = jnp.zeros((16,), jnp.float32)

    def _zero_row(i, _):
        for q in range(H // 16):
            rows_v[i, pl.ds(q * 16, 16)] = z16
        return 0
    lax.fori_loop(0, K, _zero_row, 0)
    for m in range(RPT // K):
        pltpu.sync_copy(rows_v, hp_sh.at[pl.ds(s * RPT + m * K, K)])
    pltpu.sync_copy(rows_v.at[pl.ds(0, RPT % K)],
                    hp_sh.at[pl.ds(s * RPT + (RPT // K) * K, RPT % K)])

    @pl.when(s == NS - 1)
    def _():
        pltpu.sync_copy(rows_v.at[pl.ds(0, TAIL)],
                        hp_sh.at[pl.ds(NS * RPT, TAIL)])

    def _prep_row(i, _):
        for k in range(K // 16):
            sl = pl.ds(k * 16, 16)
            dsts_v[i, sl] = dsts_v[i, sl] + coff
        return 0
    lax.fori_loop(0, CH, _prep_row, 0)

    plsc.subcore_barrier()

    def _chunk(j, _):
        pltpu.sync_copy(t2_hbm.at[dsts_v.at[j]], rows_v)
        pltpu.sync_copy(w_hbm.at[s, j], wch_v)
        for k in range(K // 16):
            w16 = wch_v[0, pl.ds(k * 16, 16)]
            for l in range(16):
                wspl = jnp.full((16,), w16[l], jnp.float32)
                i = k * 16 + l
                for q in range(H // 16):
                    sl = pl.ds(q * 16, 16)
                    rows_v[i, sl] = rows_v[i, sl] * wspl
        pltpu.sync_copy(rows_v, hp_sh.at[srcs_v.at[j]], add=True)
        return 0

    lax.fori_loop(0, CH, _chunk, 0)

    plsc.subcore_barrier()

    pltpu.sync_copy(hp_sh.at[pl.ds(s * RPT, RPT)],
                    hp_hbm.at[c, pl.ds(s * RPT, RPT)])

    @pl.when(s == NS - 1)
    def _():
        pltpu.sync_copy(hp_sh.at[pl.ds(NS * RPT, TAIL)],
                        hp_hbm.at[c, pl.ds(NS * RPT, TAIL)])


@jax.jit
def _scatter_pass(t2flat, w4, src3, dst3):
    mesh = plsc.VectorSubcoreMesh(core_axis_name="c", subcore_axis_name="s",
                                  num_cores=NC, num_subcores=NS)
    return pl.kernel(
        _scatter_body,
        out_type=jax.ShapeDtypeStruct((NC, N, H), jnp.float32),
        mesh=mesh,
        compiler_params=pltpu.CompilerParams(needs_layout_passes=False),
        scratch_types=[
            pltpu.VMEM((CH, K), jnp.int32),
            pltpu.VMEM((CH, K), jnp.int32),
            pltpu.VMEM((1, K), jnp.float32),
            pltpu.VMEM((K, H), jnp.float32),
            pltpu.VMEM_SHARED((N, H), jnp.float32),
        ],
    )(t2flat, w4, src3, dst3)


def _final_body(hp_ref, parts_ref, out_ref):
    hcat = jnp.concatenate([hp_ref[0], hp_ref[1]], axis=1)
    rsum = jnp.sum(parts_ref[...], axis=1, keepdims=True)
    rsum = jnp.where(rsum == 0.0, 1.0, rsum)
    o = jnp.maximum(hcat / rsum, 0.0)
    un = jnp.maximum(jnp.sqrt(jnp.sum(o * o, axis=1, keepdims=True)), EPS)
    th = jnp.tanh(un)
    g = o * (th / un)
    out_ref[...] = jnp.where(th > MAXNORM, g * (MAXNORM / th), g)


@jax.jit
def _finalize(hp, parts):
    grid = (N // BT,)
    return pl.pallas_call(
        _final_body,
        grid=grid,
        in_specs=[
            pl.BlockSpec((2, BT, H), lambda i: (0, i, 0)),
            pl.BlockSpec((BT, NS), lambda i: (i, 0)),
        ],
        out_specs=pl.BlockSpec((BT, D), lambda i: (i, 0)),
        out_shape=jax.ShapeDtypeStruct((N, D), jnp.float32),
    )(hp, parts)


def kernel(x, edge_index, W, a):
    t2, s1c, s2c = _prepass(x, W, a)
    t2flat = t2.reshape(NC * N, H)
    s1 = s1c.reshape(N)
    s2 = s2c.reshape(N)
    src3 = edge_index[0].reshape(NS, CH, K)
    dst3 = edge_index[1].reshape(NS, CH, K)
    w3, parts = _wsum_pass(s1, s2, src3, dst3)
    hp = _scatter_pass(t2flat, w3.reshape(NS, CH, 1, K), src3, dst3)
    return _finalize(hp, parts.T)
